# baseline (speedup 1.0000x reference)
import functools

import jax
import jax.numpy as jnp
from jax import lax
from jax.experimental import pallas as pl
from jax.experimental.pallas import tpu as pltpu
from jax.experimental.pallas import tpu_sc as plsc

N = 10000
E = 320000
D = 128
R = 8
RN = R * N

NC = 2
NS = 16
NW = NC * NS
L = 16

G = 80
EPW = E // NW
EPS = E // NS
NCH_E = EPW // G
NCH_C = EPS // G
WPS = RN // NS
ZR = 40
ZCH = N // ZR

_SC_MESH = plsc.VectorSubcoreMesh(core_axis_name="c", subcore_axis_name="s")
_SC_PARAMS = pltpu.CompilerParams(needs_layout_passes=False)


def _zero_1d(ref, n):
    z = jnp.zeros((L,), jnp.float32)

    def body(i, _):
        ref[pl.ds(i * L, L)] = z
        return 0

    lax.fori_loop(0, n // L, body, 0)
    if n % L:
        ref[pl.ds(n - L, L)] = z


_KD = 10


def _sc_weights_body(wk_hbm, we_hbm, cnt_sh, w_sh, wk_all, ones_v,
                     webuf, cbuf, wbuf, csem):
    c = lax.axis_index("c")
    s = lax.axis_index("s")

    pltpu.sync_copy(wk_hbm.at[s], wk_all)

    _zero_1d(cbuf, WPS)
    pltpu.sync_copy(cbuf, cnt_sh.at[pl.ds(s * WPS, WPS)])

    ones = jnp.ones((L,), jnp.float32)
    for j in range(G // L):
        ones_v[pl.ds(j * L, L)] = ones
    plsc.subcore_barrier()

    def count_grp(g, _):
        for j in range(_KD):
            ci = g * _KD + j
            pltpu.async_copy(ones_v, cnt_sh.at[wk_all.at[ci]], csem, add=True)
        for j in range(_KD):
            ci = g * _KD + j
            pltpu.make_async_copy(ones_v, cnt_sh.at[wk_all.at[ci]],
                                  csem).wait()
        return 0

    lax.fori_loop(0, NCH_C // _KD, count_grp, 0)
    plsc.subcore_barrier()

    pltpu.sync_copy(cnt_sh.at[pl.ds(s * WPS, WPS)], cbuf)

    def wcomp_at(o):
        v = cbuf[pl.ds(o, L)]
        wbuf[pl.ds(o, L)] = 1.0 / jnp.maximum(v, 1.0)

    def wcomp(i, _):
        wcomp_at(i * L)
        return 0

    lax.fori_loop(0, WPS // L, wcomp, 0)
    if WPS % L:
        wcomp_at(WPS - L)
    pltpu.sync_copy(wbuf, w_sh.at[pl.ds(s * WPS, WPS)])
    plsc.subcore_barrier()

    def we_grp(g, _):
        for j in range(_KD):
            ci = g * _KD + j
            row = c * NCH_E + ci
            pltpu.async_copy(w_sh.at[wk_all.at[row]], webuf.at[ci], csem)
        for j in range(_KD):
            ci = g * _KD + j
            row = c * NCH_E + ci
            pltpu.make_async_copy(w_sh.at[wk_all.at[row]], webuf.at[ci],
                                  csem).wait()
        return 0

    lax.fori_loop(0, NCH_E // _KD, we_grp, 0)
    for j in range(NCH_E % _KD):
        ci = (NCH_E // _KD) * _KD + j
        pltpu.async_copy(w_sh.at[wk_all.at[c * NCH_E + ci]], webuf.at[ci],
                         csem)
    for j in range(NCH_E % _KD):
        ci = (NCH_E // _KD) * _KD + j
        pltpu.make_async_copy(w_sh.at[wk_all.at[c * NCH_E + ci]],
                              webuf.at[ci], csem).wait()
    pltpu.sync_copy(webuf, we_hbm.at[s, c])


_sc_weights = functools.partial(
    pl.kernel,
    out_type=jax.ShapeDtypeStruct((NS, NC, NCH_E, G), jnp.float32),
    mesh=_SC_MESH,
    compiler_params=_SC_PARAMS,
    scratch_types=[
        pltpu.VMEM_SHARED((RN,), jnp.float32),
        pltpu.VMEM_SHARED((RN,), jnp.float32),
        pltpu.VMEM((NCH_C, G), jnp.int32),
        pltpu.VMEM((G,), jnp.float32),
        pltpu.VMEM((NCH_E, G), jnp.float32),
        pltpu.VMEM((WPS,), jnp.float32),
        pltpu.VMEM((WPS,), jnp.float32),
        pltpu.SemaphoreType.DMA,
    ],
)(_sc_weights_body)


def _sc_edges_body(y_hbm, gk_hbm, dst_hbm, we_hbm, out_hbm, acc_sh, rows_a,
                   rows_b, rows_c, gk_all, dst_a, dst_b, dst_c, we_a, we_b,
                   we_c, zb, gsem_a, gsem_b, gsem_c, dsem_a, dsem_b, dsem_c,
                   ssem_a, ssem_b, ssem_c):
    c = lax.axis_index("c")
    s = lax.axis_index("s")
    wid = s * NC + c
    ebase = wid * EPW

    rows = (rows_a, rows_b, rows_c)
    dstb = (dst_a, dst_b, dst_c)
    web = (we_a, we_b, we_c)
    gsem = (gsem_a, gsem_b, gsem_c)
    dsem = (dsem_a, dsem_b, dsem_c)
    ssem = (ssem_a, ssem_b, ssem_c)

    pltpu.sync_copy(gk_hbm.at[pl.ds(ebase, EPW)], gk_all)

    def zrow(j, _):
        for k in range(D // L):
            zb[j, pl.ds(k * L, L)] = jnp.zeros((L,), jnp.float32)
        return 0

    lax.fori_loop(0, ZR, zrow, 0)
    for t in range(ZCH):
        @pl.when(t % NS == s)
        def _():
            pltpu.sync_copy(zb, acc_sh.at[pl.ds(t * ZR, ZR)])
    plsc.subcore_barrier()

    def start(ci, b):
        @pl.when(ci >= 3)
        def _():
            pltpu.make_async_copy(rows[b], acc_sh.at[dstb[b]],
                                  ssem[b]).wait()
        pltpu.async_copy(y_hbm.at[gk_all.at[pl.ds(ci * G, G)]], rows[b],
                         gsem[b])
        pltpu.async_copy(dst_hbm.at[pl.ds(ebase + ci * G, G)], dstb[b],
                         dsem[b])
        pltpu.async_copy(we_hbm.at[pl.ds(ebase + ci * G, G)], web[b],
                         dsem[b])

    def process(ci, b):
        pltpu.make_async_copy(y_hbm.at[gk_all.at[pl.ds(ci * G, G)]], rows[b],
                              gsem[b]).wait()
        pltpu.make_async_copy(dst_hbm.at[pl.ds(ebase + ci * G, G)], dstb[b],
                              dsem[b]).wait()
        pltpu.make_async_copy(we_hbm.at[pl.ds(ebase + ci * G, G)], web[b],
                              dsem[b]).wait()

        def scale(j, _):
            w = plsc.load_gather(web[b],
                                 [jnp.full((L,), j, dtype=jnp.int32)])
            for k in range(D // L):
                rows[b][j, pl.ds(k * L, L)] = rows[b][j, pl.ds(k * L, L)] * w
            return 0

        lax.fori_loop(0, G, scale, 0, unroll=4)
        pltpu.async_copy(rows[b], acc_sh.at[dstb[b]], ssem[b], add=True)

    start(0, 0)
    start(1, 1)

    def triple(p, _):
        ci = 3 * p
        start(ci + 2, 2)
        process(ci, 0)
        start(ci + 3, 0)
        process(ci + 1, 1)
        start(ci + 4, 1)
        process(ci + 2, 2)
        return 0

    lax.fori_loop(0, (NCH_E - 2) // 3, triple, 0)
    process(NCH_E - 2, 0)
    process(NCH_E - 1, 1)
    for b in range(3):
        pltpu.make_async_copy(rows[b], acc_sh.at[dstb[b]], ssem[b]).wait()
    plsc.subcore_barrier()

    for t in range(ZCH):
        @pl.when(t % NS == s)
        def _():
            pltpu.sync_copy(acc_sh.at[pl.ds(t * ZR, ZR)], zb)
            pltpu.sync_copy(zb, out_hbm.at[c, pl.ds(t * ZR, ZR)])


_sc_edges = functools.partial(
    pl.kernel,
    out_type=jax.ShapeDtypeStruct((NC, N, D), jnp.float32),
    mesh=_SC_MESH,
    compiler_params=_SC_PARAMS,
    scratch_types=(
        [pltpu.VMEM_SHARED((N, D), jnp.float32)]
        + [pltpu.VMEM((G, D), jnp.float32)] * 3
        + [pltpu.VMEM((EPW,), jnp.int32)]
        + [pltpu.VMEM((G,), jnp.int32)] * 3
        + [pltpu.VMEM((G,), jnp.float32)] * 3
        + [pltpu.VMEM((ZR, D), jnp.float32)]
        + [pltpu.SemaphoreType.DMA] * 9
    ),
)(_sc_edges_body)


_EROWS = E // D


def _keys_body(src_ref, dst_ref, et_ref, gk_ref, wk_ref):
    t = et_ref[...] * N
    gk_ref[...] = t + src_ref[...]
    wk_ref[...] = t + dst_ref[...]


def _tc_keys(src, dst, et):
    return pl.pallas_call(
        _keys_body,
        out_shape=[jax.ShapeDtypeStruct((_EROWS, D), jnp.int32)] * 2,
    )(src.reshape(_EROWS, D), dst.reshape(_EROWS, D), et.reshape(_EROWS, D))


_BN = 2000


def _ymat_body(x_ref, w_ref, y_ref):
    y_ref[0] = jnp.dot(x_ref[...], w_ref[0],
                       preferred_element_type=jnp.float32)


def _tc_ymat(x, W):
    return pl.pallas_call(
        _ymat_body,
        grid=(R, N // _BN),
        in_specs=[
            pl.BlockSpec((_BN, D), lambda r, i: (i, 0)),
            pl.BlockSpec((1, D, D), lambda r, i: (r, 0, 0)),
        ],
        out_specs=pl.BlockSpec((1, _BN, D), lambda r, i: (r, i, 0)),
        out_shape=jax.ShapeDtypeStruct((R, N, D), jnp.float32),
    )(x, W)


def _hblk(x_ref, root_ref, b_ref, acc_ref):
    return jnp.maximum(
        jnp.dot(x_ref[...], root_ref[...], preferred_element_type=jnp.float32)
        + b_ref[...] + acc_ref[0] + acc_ref[1], 0.0)


def _ymat2_body(x_ref, root_ref, b_ref, acc_ref, w_ref, y_ref):
    y_ref[0] = jnp.dot(_hblk(x_ref, root_ref, b_ref, acc_ref), w_ref[0],
                       preferred_element_type=jnp.float32)


def _tc_ymat2(x, root1, b1, acc1, W2):
    return pl.pallas_call(
        _ymat2_body,
        grid=(R, N // _BN),
        in_specs=[
            pl.BlockSpec((_BN, D), lambda r, i: (i, 0)),
            pl.BlockSpec((D, D), lambda r, i: (0, 0)),
            pl.BlockSpec((1, D), lambda r, i: (0, 0)),
            pl.BlockSpec((NC, _BN, D), lambda r, i: (0, i, 0)),
            pl.BlockSpec((1, D, D), lambda r, i: (r, 0, 0)),
        ],
        out_specs=pl.BlockSpec((1, _BN, D), lambda r, i: (r, i, 0)),
        out_shape=jax.ShapeDtypeStruct((R, N, D), jnp.float32),
    )(x, root1, b1.reshape(1, D), acc1, W2)


def _combine2_body(x_ref, root1_ref, b1_ref, acc1_ref, root2_ref, b2_ref,
                   acc2_ref, o_ref):
    h = _hblk(x_ref, root1_ref, b1_ref, acc1_ref)
    o_ref[...] = (jnp.dot(h, root2_ref[...],
                          preferred_element_type=jnp.float32)
                  + b2_ref[...] + acc2_ref[0] + acc2_ref[1])


def _tc_combine2(x, root1, b1, acc1, root2, b2, acc2):
    return pl.pallas_call(
        _combine2_body,
        grid=(N // _BN,),
        in_specs=[
            pl.BlockSpec((_BN, D), lambda i: (i, 0)),
            pl.BlockSpec((D, D), lambda i: (0, 0)),
            pl.BlockSpec((1, D), lambda i: (0, 0)),
            pl.BlockSpec((NC, _BN, D), lambda i: (0, i, 0)),
            pl.BlockSpec((D, D), lambda i: (0, 0)),
            pl.BlockSpec((1, D), lambda i: (0, 0)),
            pl.BlockSpec((NC, _BN, D), lambda i: (0, i, 0)),
        ],
        out_specs=pl.BlockSpec((_BN, D), lambda i: (i, 0)),
        out_shape=jax.ShapeDtypeStruct((N, D), jnp.float32),
    )(x, root1, b1.reshape(1, D), acc1, root2, b2.reshape(1, D), acc2)


def kernel(x, edge_index, edge_type, W1, root1, b1, W2, root2, b2):
    src = edge_index[0]
    dst = edge_index[1]

    gk2d, wk2d = _tc_keys(src, dst, edge_type)
    gk = gk2d.reshape(E)
    wk = wk2d.reshape(E)

    we = _sc_weights(wk.reshape(NS, NCH_C, G)).reshape(E)

    y1 = _tc_ymat(x, W1).reshape(RN, D)
    acc1 = _sc_edges(y1, gk, dst, we)

    y2 = _tc_ymat2(x, root1, b1, acc1, W2).reshape(RN, D)
    acc2 = _sc_edges(y2, gk, dst, we)
    return _tc_combine2(x, root1, b1, acc1, root2, b2, acc2)


# speedup vs baseline: 1.0598x; 1.0598x over previous
import functools

import jax
import jax.numpy as jnp
from jax import lax
from jax.experimental import pallas as pl
from jax.experimental.pallas import tpu as pltpu
from jax.experimental.pallas import tpu_sc as plsc

N = 10000
E = 320000
D = 128
R = 8
RN = R * N

NC = 2
NS = 16
NW = NC * NS
L = 16

G = 80
EPW = E // NW
EPS = E // NS
NCH_E = EPW // G
NCH_C = EPS // G
WPS = RN // NS
ZR = 40
ZCH = N // ZR

_SC_MESH = plsc.VectorSubcoreMesh(core_axis_name="c", subcore_axis_name="s")
_SC_PARAMS = pltpu.CompilerParams(needs_layout_passes=False)


def _zero_1d(ref, n):
    z = jnp.zeros((L,), jnp.float32)

    def body(i, _):
        ref[pl.ds(i * L, L)] = z
        return 0

    lax.fori_loop(0, n // L, body, 0)
    if n % L:
        ref[pl.ds(n - L, L)] = z


_KD = 10


def _sc_weights_body(wk_hbm, we_hbm, cnt_sh, w_sh, wk_all, ones_v,
                     webuf, cbuf, wbuf, csem):
    c = lax.axis_index("c")
    s = lax.axis_index("s")

    pltpu.sync_copy(wk_hbm.at[s], wk_all)

    _zero_1d(cbuf, WPS)
    pltpu.sync_copy(cbuf, cnt_sh.at[pl.ds(s * WPS, WPS)])

    ones = jnp.ones((L,), jnp.float32)
    for j in range(G // L):
        ones_v[pl.ds(j * L, L)] = ones
    plsc.subcore_barrier()

    def count_grp(g, _):
        for j in range(_KD):
            ci = g * _KD + j
            pltpu.async_copy(ones_v, cnt_sh.at[wk_all.at[ci]], csem, add=True)
        for j in range(_KD):
            ci = g * _KD + j
            pltpu.make_async_copy(ones_v, cnt_sh.at[wk_all.at[ci]],
                                  csem).wait()
        return 0

    lax.fori_loop(0, NCH_C // _KD, count_grp, 0)
    plsc.subcore_barrier()

    pltpu.sync_copy(cnt_sh.at[pl.ds(s * WPS, WPS)], cbuf)

    def wcomp_at(o):
        v = cbuf[pl.ds(o, L)]
        wbuf[pl.ds(o, L)] = 1.0 / jnp.maximum(v, 1.0)

    def wcomp(i, _):
        wcomp_at(i * L)
        return 0

    lax.fori_loop(0, WPS // L, wcomp, 0)
    if WPS % L:
        wcomp_at(WPS - L)
    pltpu.sync_copy(wbuf, w_sh.at[pl.ds(s * WPS, WPS)])
    plsc.subcore_barrier()

    def we_grp(g, _):
        for j in range(_KD):
            ci = g * _KD + j
            row = c * NCH_E + ci
            pltpu.async_copy(w_sh.at[wk_all.at[row]], webuf.at[ci], csem)
        for j in range(_KD):
            ci = g * _KD + j
            row = c * NCH_E + ci
            pltpu.make_async_copy(w_sh.at[wk_all.at[row]], webuf.at[ci],
                                  csem).wait()
        return 0

    lax.fori_loop(0, NCH_E // _KD, we_grp, 0)
    for j in range(NCH_E % _KD):
        ci = (NCH_E // _KD) * _KD + j
        pltpu.async_copy(w_sh.at[wk_all.at[c * NCH_E + ci]], webuf.at[ci],
                         csem)
    for j in range(NCH_E % _KD):
        ci = (NCH_E // _KD) * _KD + j
        pltpu.make_async_copy(w_sh.at[wk_all.at[c * NCH_E + ci]],
                              webuf.at[ci], csem).wait()
    pltpu.sync_copy(webuf, we_hbm.at[s, c])


_sc_weights = functools.partial(
    pl.kernel,
    out_type=jax.ShapeDtypeStruct((NS, NC, NCH_E, G), jnp.float32),
    mesh=_SC_MESH,
    compiler_params=_SC_PARAMS,
    scratch_types=[
        pltpu.VMEM_SHARED((RN,), jnp.float32),
        pltpu.VMEM_SHARED((RN,), jnp.float32),
        pltpu.VMEM((NCH_C, G), jnp.int32),
        pltpu.VMEM((G,), jnp.float32),
        pltpu.VMEM((NCH_E, G), jnp.float32),
        pltpu.VMEM((WPS,), jnp.float32),
        pltpu.VMEM((WPS,), jnp.float32),
        pltpu.SemaphoreType.DMA,
    ],
)(_sc_weights_body)


def _sc_edges_body(y_hbm, gk_hbm, dst_hbm, we_hbm, out_hbm, acc_sh, rows_a,
                   rows_b, rows_c, gk_all, dst_a, dst_b, dst_c, we_a, we_b,
                   we_c, zb, gsem_a, gsem_b, gsem_c, dsem_a, dsem_b, dsem_c,
                   ssem_a, ssem_b, ssem_c):
    c = lax.axis_index("c")
    s = lax.axis_index("s")
    wid = s * NC + c
    ebase = wid * EPW

    rows = (rows_a, rows_b, rows_c)
    dstb = (dst_a, dst_b, dst_c)
    web = (we_a, we_b, we_c)
    gsem = (gsem_a, gsem_b, gsem_c)
    dsem = (dsem_a, dsem_b, dsem_c)
    ssem = (ssem_a, ssem_b, ssem_c)

    pltpu.sync_copy(gk_hbm.at[pl.ds(ebase, EPW)], gk_all)

    def zrow(j, _):
        for k in range(D // L):
            zb[j, pl.ds(k * L, L)] = jnp.zeros((L,), jnp.float32)
        return 0

    lax.fori_loop(0, ZR, zrow, 0)
    for t in range(ZCH):
        @pl.when(t % NS == s)
        def _():
            pltpu.sync_copy(zb, acc_sh.at[pl.ds(t * ZR, ZR)])
    plsc.subcore_barrier()

    def start(ci, b):
        @pl.when(ci >= 3)
        def _():
            pltpu.make_async_copy(rows[b], acc_sh.at[dstb[b]],
                                  ssem[b]).wait()
        pltpu.async_copy(y_hbm.at[gk_all.at[pl.ds(ci * G, G)]], rows[b],
                         gsem[b])
        pltpu.async_copy(dst_hbm.at[pl.ds(ebase + ci * G, G)], dstb[b],
                         dsem[b])
        pltpu.async_copy(we_hbm.at[pl.ds(ebase + ci * G, G)], web[b],
                         dsem[b])

    def process(ci, b):
        pltpu.make_async_copy(y_hbm.at[gk_all.at[pl.ds(ci * G, G)]], rows[b],
                              gsem[b]).wait()
        pltpu.make_async_copy(dst_hbm.at[pl.ds(ebase + ci * G, G)], dstb[b],
                              dsem[b]).wait()
        pltpu.make_async_copy(we_hbm.at[pl.ds(ebase + ci * G, G)], web[b],
                              dsem[b]).wait()

        def scale(j, _):
            w = plsc.load_gather(web[b],
                                 [jnp.full((L,), j, dtype=jnp.int32)])
            for k in range(D // L):
                rows[b][j, pl.ds(k * L, L)] = rows[b][j, pl.ds(k * L, L)] * w
            return 0

        lax.fori_loop(0, G, scale, 0, unroll=4)
        pltpu.async_copy(rows[b], acc_sh.at[dstb[b]], ssem[b], add=True)

    start(0, 0)
    start(1, 1)

    def triple(p, _):
        ci = 3 * p
        start(ci + 2, 2)
        process(ci, 0)
        start(ci + 3, 0)
        process(ci + 1, 1)
        start(ci + 4, 1)
        process(ci + 2, 2)
        return 0

    lax.fori_loop(0, (NCH_E - 2) // 3, triple, 0)
    process(NCH_E - 2, 0)
    process(NCH_E - 1, 1)
    for b in range(3):
        pltpu.make_async_copy(rows[b], acc_sh.at[dstb[b]], ssem[b]).wait()
    plsc.subcore_barrier()

    for t in range(ZCH):
        @pl.when(t % NS == s)
        def _():
            pltpu.sync_copy(acc_sh.at[pl.ds(t * ZR, ZR)], zb)
            pltpu.sync_copy(zb, out_hbm.at[c, pl.ds(t * ZR, ZR)])


_sc_edges = functools.partial(
    pl.kernel,
    out_type=jax.ShapeDtypeStruct((NC, N, D), jnp.float32),
    mesh=_SC_MESH,
    compiler_params=_SC_PARAMS,
    scratch_types=(
        [pltpu.VMEM_SHARED((N, D), jnp.float32)]
        + [pltpu.VMEM((G, D), jnp.float32)] * 3
        + [pltpu.VMEM((EPW,), jnp.int32)]
        + [pltpu.VMEM((G,), jnp.int32)] * 3
        + [pltpu.VMEM((G,), jnp.float32)] * 3
        + [pltpu.VMEM((ZR, D), jnp.float32)]
        + [pltpu.SemaphoreType.DMA] * 9
    ),
)(_sc_edges_body)


_EROWS = E // D


def _keys_body(src_ref, dst_ref, et_ref, gk_ref, wk_ref):
    t = et_ref[...] * N
    gk_ref[...] = t + src_ref[...]
    wk_ref[...] = t + dst_ref[...]


def _tc_keys(src, dst, et):
    return pl.pallas_call(
        _keys_body,
        out_shape=[jax.ShapeDtypeStruct((_EROWS, D), jnp.int32)] * 2,
    )(src.reshape(_EROWS, D), dst.reshape(_EROWS, D), et.reshape(_EROWS, D))


_BN = 2000


def _ymat_body(x_ref, w_ref, y_ref):
    y_ref[0] = jnp.dot(x_ref[...], w_ref[0],
                       preferred_element_type=jnp.float32)


def _tc_ymat(x, W):
    return pl.pallas_call(
        _ymat_body,
        grid=(N // _BN, R),
        in_specs=[
            pl.BlockSpec((_BN, D), lambda i, r: (i, 0)),
            pl.BlockSpec((1, D, D), lambda i, r: (r, 0, 0)),
        ],
        out_specs=pl.BlockSpec((1, _BN, D), lambda i, r: (r, i, 0)),
        out_shape=jax.ShapeDtypeStruct((R, N, D), jnp.float32),
    )(x, W)


def _hblk(x_ref, root_ref, b_ref, acc_ref):
    return jnp.maximum(
        jnp.dot(x_ref[...], root_ref[...], preferred_element_type=jnp.float32)
        + b_ref[...] + acc_ref[0] + acc_ref[1], 0.0)


def _ymat2_body(x_ref, root_ref, b_ref, acc_ref, w_ref, y_ref):
    y_ref[0] = jnp.dot(_hblk(x_ref, root_ref, b_ref, acc_ref), w_ref[0],
                       preferred_element_type=jnp.float32)


def _tc_ymat2(x, root1, b1, acc1, W2):
    return pl.pallas_call(
        _ymat2_body,
        grid=(N // _BN, R),
        in_specs=[
            pl.BlockSpec((_BN, D), lambda i, r: (i, 0)),
            pl.BlockSpec((D, D), lambda i, r: (0, 0)),
            pl.BlockSpec((1, D), lambda i, r: (0, 0)),
            pl.BlockSpec((NC, _BN, D), lambda i, r: (0, i, 0)),
            pl.BlockSpec((1, D, D), lambda i, r: (r, 0, 0)),
        ],
        out_specs=pl.BlockSpec((1, _BN, D), lambda i, r: (r, i, 0)),
        out_shape=jax.ShapeDtypeStruct((R, N, D), jnp.float32),
    )(x, root1, b1.reshape(1, D), acc1, W2)


def _combine2_body(x_ref, root1_ref, b1_ref, acc1_ref, root2_ref, b2_ref,
                   acc2_ref, o_ref):
    h = _hblk(x_ref, root1_ref, b1_ref, acc1_ref)
    o_ref[...] = (jnp.dot(h, root2_ref[...],
                          preferred_element_type=jnp.float32)
                  + b2_ref[...] + acc2_ref[0] + acc2_ref[1])


def _tc_combine2(x, root1, b1, acc1, root2, b2, acc2):
    return pl.pallas_call(
        _combine2_body,
        grid=(N // _BN,),
        in_specs=[
            pl.BlockSpec((_BN, D), lambda i: (i, 0)),
            pl.BlockSpec((D, D), lambda i: (0, 0)),
            pl.BlockSpec((1, D), lambda i: (0, 0)),
            pl.BlockSpec((NC, _BN, D), lambda i: (0, i, 0)),
            pl.BlockSpec((D, D), lambda i: (0, 0)),
            pl.BlockSpec((1, D), lambda i: (0, 0)),
            pl.BlockSpec((NC, _BN, D), lambda i: (0, i, 0)),
        ],
        out_specs=pl.BlockSpec((_BN, D), lambda i: (i, 0)),
        out_shape=jax.ShapeDtypeStruct((N, D), jnp.float32),
    )(x, root1, b1.reshape(1, D), acc1, root2, b2.reshape(1, D), acc2)


def kernel(x, edge_index, edge_type, W1, root1, b1, W2, root2, b2):
    src = edge_index[0]
    dst = edge_index[1]

    gk2d, wk2d = _tc_keys(src, dst, edge_type)
    gk = gk2d.reshape(E)
    wk = wk2d.reshape(E)

    we = _sc_weights(wk.reshape(NS, NCH_C, G)).reshape(E)

    y1 = _tc_ymat(x, W1).reshape(RN, D)
    acc1 = _sc_edges(y1, gk, dst, we)

    y2 = _tc_ymat2(x, root1, b1, acc1, W2).reshape(RN, D)
    acc2 = _sc_edges(y2, gk, dst, we)
    return _tc_combine2(x, root1, b1, acc1, root2, b2, acc2)
